# baseline (speedup 1.0000x reference)
import functools

import jax
import jax.numpy as jnp
from jax import lax
from jax.experimental import pallas as pl
from jax.experimental.pallas import tpu as pltpu
from jax.experimental.pallas import tpu_sc as plsc

D_MODEL = 1024
SCALE = 32.0

_info = plsc.get_sparse_core_info()
NUM_CORES = _info.num_cores
NUM_SUBCORES = _info.num_subcores
NUM_WORKERS = NUM_CORES * NUM_SUBCORES
LANES = _info.num_lanes

B_TOTAL = 4 * 4096
B_PER_W = B_TOTAL // NUM_WORKERS
CHUNK = 16
NSTEPS = B_PER_W // CHUNK
SL_PER_ROW = D_MODEL // LANES
UNROLL = 8
NBUF = 5
AHEAD = 4


def _scale_buf(buf):
    @plsc.parallel_loop(0, CHUNK * SL_PER_ROW, step=1, unroll=UNROLL)
    def _(k):
        row = k // SL_PER_ROW
        col = (k % SL_PER_ROW) * LANES
        sl = pl.ds(col, LANES)
        buf[row, sl] = buf[row, sl] * SCALE


NS = 2


def _embed_body(idx_hbm, table_hbm, out_hbm, idx_v, bufs, spm, gsems, csems,
                dsems):
    sid = lax.axis_index("s")
    wid = sid * NUM_CORES + lax.axis_index("c")
    base = wid * B_PER_W
    pltpu.sync_copy(idx_hbm.at[pl.ds(base, B_PER_W)], idx_v)

    gh = [None] * NSTEPS
    ch = [None] * NSTEPS
    dh = [None] * NSTEPS

    def start_gather(s):
        b = s % NBUF
        idx_sl = idx_v.at[pl.ds(s * CHUNK, CHUNK)]
        gh[s] = pltpu.async_copy(table_hbm.at[idx_sl], bufs.at[b], gsems.at[b])

    def start_dma(s):
        dh[s] = pltpu.async_copy(
            spm.at[sid, s % NS],
            out_hbm.at[pl.ds(base + s * CHUNK, CHUNK)],
            dsems.at[s % NS],
        )

    for s in range(AHEAD):
        start_gather(s)
    for s in range(NSTEPS):
        b = s % NBUF
        gh[s].wait()
        if s >= 1:
            ch[s - 1].wait()
            start_dma(s - 1)
        if s + AHEAD < NSTEPS:
            start_gather(s + AHEAD)
        _scale_buf(bufs.at[b])
        if s >= NS:
            dh[s - NS].wait()
        ch[s] = pltpu.async_copy(bufs.at[b], spm.at[sid, s % NS], csems.at[b])
    ch[NSTEPS - 1].wait()
    start_dma(NSTEPS - 1)
    for s in range(NSTEPS - NS, NSTEPS):
        dh[s].wait()


@jax.jit
def _embed(x_flat, table):
    mesh = plsc.VectorSubcoreMesh(core_axis_name="c", subcore_axis_name="s")
    fn = pl.kernel(
        _embed_body,
        out_type=jax.ShapeDtypeStruct((B_TOTAL, D_MODEL), jnp.float32),
        mesh=mesh,
        scratch_types=[
            pltpu.VMEM((B_PER_W,), jnp.int32),
            pltpu.VMEM((NBUF, CHUNK, D_MODEL), jnp.float32),
            pltpu.VMEM_SHARED((NUM_SUBCORES, NS, CHUNK, D_MODEL), jnp.float32),
            pltpu.SemaphoreType.DMA((NBUF,)),
            pltpu.SemaphoreType.DMA((NBUF,)),
            pltpu.SemaphoreType.DMA((NS,)),
        ],
    )
    return fn(x_flat, table)


def kernel(x, input_embedding_table_VD):
    B, T = x.shape
    x_flat = x.reshape(B * T).astype(jnp.int32)
    out = _embed(x_flat, input_embedding_table_VD)
    return out.reshape(B, T, D_MODEL)


# speedup vs baseline: 1.0612x; 1.0612x over previous
import functools

import jax
import jax.numpy as jnp
from jax import lax
from jax.experimental import pallas as pl
from jax.experimental.pallas import tpu as pltpu
from jax.experimental.pallas import tpu_sc as plsc

D_MODEL = 1024
SCALE = 32.0

_info = plsc.get_sparse_core_info()
NUM_CORES = _info.num_cores
NUM_SUBCORES = _info.num_subcores
NUM_WORKERS = NUM_CORES * NUM_SUBCORES
LANES = _info.num_lanes

B_TOTAL = 4 * 4096
B_PER_W = B_TOTAL // NUM_WORKERS
CHUNK = 16
NSTEPS = B_PER_W // CHUNK
SL_PER_ROW = D_MODEL // LANES
UNROLL = 8
NBUF = 7
AHEAD = 6


def _scale_buf(buf):
    @plsc.parallel_loop(0, CHUNK * SL_PER_ROW, step=1, unroll=UNROLL)
    def _(k):
        row = k // SL_PER_ROW
        col = (k % SL_PER_ROW) * LANES
        sl = pl.ds(col, LANES)
        buf[row, sl] = buf[row, sl] * SCALE


def _embed_body(idx_hbm, table_hbm, out_hbm, idx_v, bufs, gsems, ssems):
    wid = lax.axis_index("s") * NUM_CORES + lax.axis_index("c")
    base = wid * B_PER_W
    pltpu.sync_copy(idx_hbm.at[pl.ds(base, B_PER_W)], idx_v)

    gh = [None] * NSTEPS
    sh = [None] * NSTEPS

    def start_gather(s):
        b = s % NBUF
        idx_sl = idx_v.at[pl.ds(s * CHUNK, CHUNK)]
        gh[s] = pltpu.async_copy(table_hbm.at[idx_sl], bufs.at[b], gsems.at[b])

    for s in range(AHEAD):
        start_gather(s)
    for s in range(NSTEPS):
        b = s % NBUF
        gh[s].wait()
        if s + AHEAD < NSTEPS:
            ps = s + AHEAD - NBUF
            if ps >= 0:
                sh[ps].wait()
            start_gather(s + AHEAD)
        _scale_buf(bufs.at[b])
        sh[s] = pltpu.async_copy(
            bufs.at[b], out_hbm.at[pl.ds(base + s * CHUNK, CHUNK)], ssems.at[b]
        )
    for s in range(NSTEPS - NBUF, NSTEPS):
        sh[s].wait()


@jax.jit
def _embed(x_flat, table):
    mesh = plsc.VectorSubcoreMesh(core_axis_name="c", subcore_axis_name="s")
    fn = pl.kernel(
        _embed_body,
        out_type=jax.ShapeDtypeStruct((B_TOTAL, D_MODEL), jnp.float32),
        mesh=mesh,
        scratch_types=[
            pltpu.VMEM((B_PER_W,), jnp.int32),
            pltpu.VMEM((NBUF, CHUNK, D_MODEL), jnp.float32),
            pltpu.SemaphoreType.DMA((NBUF,)),
            pltpu.SemaphoreType.DMA((NBUF,)),
        ],
    )
    return fn(x_flat, table)


def kernel(x, input_embedding_table_VD):
    B, T = x.shape
    x_flat = x.reshape(B * T).astype(jnp.int32)
    out = _embed(x_flat, input_embedding_table_VD)
    return out.reshape(B, T, D_MODEL)
